# baseline (speedup 1.0000x reference)
import functools
import numpy as np
import jax
import jax.numpy as jnp
from jax import lax
from jax.experimental import pallas as pl
from jax.experimental.pallas import tpu as pltpu
from jax.experimental.pallas import tpu_sc as plsc

R_CUTOFF = 5.0
NUM_RBF = 6
NUM_SBF = 7
EMBED = 128
N_SPECIES = 20
NUM_TARGETS = 1
N_INTER = 4
ENV_P = 6
BASIS_EMB = 8
ANGLE_EMB = 64
TYPE_EMB = 64
OUT_EMB = 256
N_PART = 10000
N_EDGES = 160000
N_ANGLES = 320000

NC, NS = 2, 16
NW = NC * NS
E_PER_W = N_EDGES // NW
SUB = 5
E_SUB = E_PER_W // SUB
NSUB = NW * SUB
T_PER_W = 320
N_TPAD = NW * T_PER_W

C_TRI = 20
A2_CAP = NSUB * C_TRI * 128
A2_CH = A2_CAP // (NW * 128)
C_AT = 48
E2_CAP = NW * C_AT * 128
EG_CAP = 163840
EG_CH = 40

B_E = 1280
B_A = 1024
B_T = 1000


def _sph_jl_np(l, x):
    x = np.asarray(x, dtype=np.float64)
    j0 = np.sin(x) / x
    if l == 0:
        return j0
    j1 = np.sin(x) / x**2 - np.cos(x) / x
    if l == 1:
        return j1
    jm, jc = j0, j1
    for i in range(1, l):
        jn = (2 * i + 1) / x * jc - jm
        jm, jc = jc, jn
    return jc


def _bessel_zeros(num_l, num_n):
    zeros = np.zeros((num_l, num_n))
    xs = np.linspace(1e-2, 80.0, 160001)
    for l in range(num_l):
        vals = _sph_jl_np(l, xs)
        s = np.sign(vals)
        idx = np.where(s[:-1] * s[1:] < 0)[0][:num_n]
        for n, i in enumerate(idx):
            a, b = xs[i], xs[i + 1]
            fa = _sph_jl_np(l, np.array([a]))[0]
            for _ in range(60):
                mid = 0.5 * (a + b)
                fm = _sph_jl_np(l, np.array([mid]))[0]
                if fa * fm <= 0:
                    b = mid
                else:
                    a, fa = mid, fm
            zeros[l, n] = 0.5 * (a + b)
    return zeros


_ZEROS64 = _bessel_zeros(NUM_SBF, NUM_RBF)
_NORM64 = np.zeros((NUM_SBF, NUM_RBF))
for _l in range(NUM_SBF):
    _NORM64[_l] = (np.sqrt(2.0 / R_CUTOFF**3)
                   / np.abs(_sph_jl_np(_l + 1, _ZEROS64[_l])))
_ZEROS_NP = _ZEROS64.astype(np.float32)
_NORM_NP = _NORM64.astype(np.float32)
_LEG_COEF = np.array([np.sqrt((2 * l + 1) / (4 * np.pi)) for l in range(NUM_SBF)],
                     np.float32)


def _act(x):
    return x * jax.nn.sigmoid(x)


def _envelope(x):
    p = ENV_P + 1
    a = -(p + 1) * (p + 2) / 2.0
    b = p * (p + 2)
    c = -p * (p + 1) / 2.0
    x2 = x * x
    x4 = x2 * x2
    x6 = x4 * x2
    env = 1.0 / x + a * x6 + b * x6 * x + c * x4 * x4
    return jnp.where(x < 1.0, env, 0.0)


def _rbf_block(d):
    x = d / R_CUTOFF
    env = _envelope(x)
    cols = [jnp.sin(float(np.float32(np.pi * (n + 1))) * x)
            for n in range(NUM_RBF)]
    return env[:, None] * jnp.stack(cols, axis=1)


def _sph_jl(l, x):
    j0 = jnp.sin(x) / x
    if l == 0:
        return j0
    j1 = jnp.sin(x) / x**2 - jnp.cos(x) / x
    if l == 1:
        return j1
    jm, jc = j0, j1
    for i in range(1, l):
        jn = (2 * i + 1) / x * jc - jm
        jm, jc = jc, jn
    return jc


def _rad_body(d_ref, o_ref):
    d = d_ref[0, 0]
    x = d / R_CUTOFF
    env = _envelope(x)
    cols = []
    for l in range(NUM_SBF):
        for n in range(NUM_RBF):
            cols.append(_sph_jl(l, x * float(_ZEROS_NP[l, n]))
                        * float(_NORM_NP[l, n]))
    r42 = jnp.stack(cols, axis=1) * env[:, None]
    o_ref[...] = jnp.concatenate(
        [r42, jnp.zeros((r42.shape[0], EMBED - 42), jnp.float32)], axis=1)


def _k_rad128(d3):
    grid = (N_EDGES // B_E,)
    return pl.pallas_call(
        _rad_body, grid=grid,
        in_specs=[pl.BlockSpec((1, 1, B_E), lambda i: (i, 0, 0))],
        out_specs=pl.BlockSpec((B_E, EMBED), lambda i: (i, 0)),
        out_shape=jax.ShapeDtypeStruct((N_EDGES, EMBED), jnp.float32),
    )(d3)


B_CA = 1280


def _cbf_body(ang_ref, mask_ref, o_ref):
    ang = ang_ref[0, 0]
    maskc = mask_ref[0, 0]
    cos_t = jnp.cos(ang)
    ps = [jnp.ones_like(cos_t), cos_t]
    for l in range(1, NUM_SBF - 1):
        ps.append(((2 * l + 1) * cos_t * ps[l] - l * ps[l - 1]) / (l + 1))
    cols = []
    for l in range(NUM_SBF):
        cl = float(_LEG_COEF[l]) * ps[l] * maskc
        cols.extend([cl] * NUM_RBF)
    c42 = jnp.stack(cols, axis=1)
    o_ref[...] = jnp.concatenate(
        [c42, jnp.zeros((c42.shape[0], EMBED - 42), jnp.float32)], axis=1)


def _k_cbf128(ang3, mask3):
    grid = (N_ANGLES // B_CA,)
    bs1 = pl.BlockSpec((1, 1, B_CA), lambda i: (i, 0, 0))
    return pl.pallas_call(
        _cbf_body, grid=grid,
        in_specs=[bs1, bs1],
        out_specs=pl.BlockSpec((B_CA, EMBED), lambda i: (i, 0)),
        out_shape=jax.ShapeDtypeStruct((N_ANGLES, EMBED), jnp.float32),
    )(ang3, mask3)


B_SP = 1000


def _ctab_body(sp_ref, embwj_ref, embwi_ref, cj_ref, ci_ref):
    ids = lax.broadcasted_iota(jnp.int32, (B_SP, N_SPECIES), 1)
    oh = (sp_ref[0, 0][:, None] == ids).astype(jnp.float32)
    cj_ref[...] = jnp.dot(oh, embwj_ref[...], preferred_element_type=jnp.float32)
    ci_ref[...] = jnp.dot(oh, embwi_ref[...], preferred_element_type=jnp.float32)


def _k_ctab(sp3, embwj, embwi):
    grid = (N_PART // B_SP,)
    rep = lambda s: pl.BlockSpec(s, lambda i: (0,) * len(s))
    return pl.pallas_call(
        _ctab_body, grid=grid,
        in_specs=[pl.BlockSpec((1, 1, B_SP), lambda i: (i, 0, 0)),
                  rep((N_SPECIES, EMBED)), rep((N_SPECIES, EMBED))],
        out_specs=[pl.BlockSpec((B_SP, EMBED), lambda i: (i, 0))] * 2,
        out_shape=[jax.ShapeDtypeStruct((N_PART, EMBED), jnp.float32)] * 2,
    )(sp3, embwj, embwi)


def _embed_body(d_ref, cgj_ref, cgi_ref, wr_ref, be_ref,
                wre_ref, bre_ref, wout0_ref, m_ref, gm_ref):
    d = d_ref[0, 0]
    rbf = _rbf_block(d)
    rbf_e = _act(jnp.dot(rbf, wre_ref[...], preferred_element_type=jnp.float32)
                 + bre_ref[...][None, :])
    pre = (cgj_ref[...] + cgi_ref[...]
           + jnp.dot(rbf_e, wr_ref[...], preferred_element_type=jnp.float32)
           + be_ref[...][None, :])
    m = _act(pre)
    m_ref[...] = m
    gm_ref[...] = jnp.dot(rbf, wout0_ref[...],
                          preferred_element_type=jnp.float32) * m


def _k_embed(d3, cgj, cgi, wr, be, wre, bre, wout0):
    grid = (N_EDGES // B_E,)
    bs1 = pl.BlockSpec((1, 1, B_E), lambda i: (i, 0, 0))
    bsm = pl.BlockSpec((B_E, EMBED), lambda i: (i, 0))
    rep2 = lambda s: pl.BlockSpec(s, lambda i: (0,) * len(s))
    return pl.pallas_call(
        _embed_body, grid=grid,
        in_specs=[bs1, bsm, bsm,
                  rep2((EMBED, EMBED)), rep2((EMBED,)), rep2((NUM_RBF, EMBED)),
                  rep2((EMBED,)), rep2((NUM_RBF, EMBED))],
        out_specs=[bsm] * 2,
        out_shape=[jax.ShapeDtypeStruct((N_EDGES, EMBED), jnp.float32)] * 2,
    )(d3, cgj, cgi, wr, be, wre, bre, wout0)


def _pre_body(d_ref, m_ref, wji_ref, bji_ref, wkj_ref, bkj_ref, wrbf_ref,
              wdown_ref, xji_ref, xdown_ref):
    d = d_ref[0, 0]
    m = m_ref[...]
    rbf = _rbf_block(d)
    xji_ref[...] = _act(jnp.dot(m, wji_ref[...], preferred_element_type=jnp.float32)
                        + bji_ref[...][None, :])
    xkj = _act(jnp.dot(m, wkj_ref[...], preferred_element_type=jnp.float32)
               + bkj_ref[...][None, :])
    xkj = xkj * jnp.dot(rbf, wrbf_ref[...], preferred_element_type=jnp.float32)
    xd = _act(jnp.dot(xkj, wdown_ref[...], preferred_element_type=jnp.float32))
    xdown_ref[...] = jnp.concatenate(
        [xd, jnp.zeros((xd.shape[0], EMBED - ANGLE_EMB), jnp.float32)], axis=1)


def _k_pre(d3, m, wji, bji, wkj, bkj, wrbf12, wdown):
    grid = (N_EDGES // B_E,)
    bs1 = pl.BlockSpec((1, 1, B_E), lambda i: (i, 0, 0))
    bsm = pl.BlockSpec((B_E, EMBED), lambda i: (i, 0))
    rep = lambda s: pl.BlockSpec(s, lambda i: (0,) * len(s))
    return pl.pallas_call(
        _pre_body, grid=grid,
        in_specs=[bs1, bsm, rep((EMBED, EMBED)), rep((EMBED,)),
                  rep((EMBED, EMBED)), rep((EMBED,)), rep((NUM_RBF, EMBED)),
                  rep((EMBED, ANGLE_EMB))],
        out_specs=[bsm, bsm],
        out_shape=[jax.ShapeDtypeStruct((N_EDGES, EMBED), jnp.float32),
                   jax.ShapeDtypeStruct((N_EDGES, EMBED), jnp.float32)],
    )(d3, m, wji, bji, wkj, bkj, wrbf12, wdown)


def _t_body(radg_ref, cbfg_ref, xg_ref, w48_ref, t_ref):
    sbf = radg_ref[...][:, :48] * cbfg_ref[...][:, :48]
    s = jnp.dot(sbf, w48_ref[...], preferred_element_type=jnp.float32)
    t_ref[...] = s * xg_ref[...][:, :ANGLE_EMB]


def _k_t(rad_g, cbf_g, x_g, w48):
    grid = (A2_CAP // B_A,)
    bsm = pl.BlockSpec((B_A, EMBED), lambda i: (i, 0))
    return pl.pallas_call(
        _t_body, grid=grid,
        in_specs=[bsm, bsm, bsm,
                  pl.BlockSpec((48, ANGLE_EMB), lambda i: (0, 0))],
        out_specs=pl.BlockSpec((B_A, ANGLE_EMB), lambda i: (i, 0)),
        out_shape=jax.ShapeDtypeStruct((A2_CAP, ANGLE_EMB), jnp.float32),
    )(rad_g, cbf_g, x_g, w48)


def _post_body(d_ref, agg_ref, xji_ref, m_ref, wup_ref,
               rb_w1, rb_b1, rb_w2, rb_b2, wskip_ref, bskip_ref,
               ra1_w1, ra1_b1, ra1_w2, ra1_b2,
               ra2_w1, ra2_b1, ra2_w2, ra2_b2,
               wnext_ref, mnew_ref, gm_ref):
    def dot(a, w):
        return jnp.dot(a, w[...], preferred_element_type=jnp.float32)

    def res(h, w1, b1, w2, b2):
        return h + _act(dot(_act(dot(h, w1) + b1[...][None, :]), w2)
                        + b2[...][None, :])

    xkj = _act(dot(agg_ref[...], wup_ref))
    h = xji_ref[...] + xkj
    h = res(h, rb_w1, rb_b1, rb_w2, rb_b2)
    h = _act(dot(h, wskip_ref) + bskip_ref[...][None, :]) + m_ref[...]
    h = res(h, ra1_w1, ra1_b1, ra1_w2, ra1_b2)
    h = res(h, ra2_w1, ra2_b1, ra2_w2, ra2_b2)
    mnew_ref[...] = h
    rbf = _rbf_block(d_ref[0, 0])
    gm_ref[...] = dot(rbf, wnext_ref) * h


def _k_post(d3, agg, xji, m, wup, rb, wskip, bskip, ra1, ra2, wnext):
    grid = (N_EDGES // B_E,)
    bs1 = pl.BlockSpec((1, 1, B_E), lambda i: (i, 0, 0))
    bsm = pl.BlockSpec((B_E, EMBED), lambda i: (i, 0))
    bsa = pl.BlockSpec((B_E, ANGLE_EMB), lambda i: (i, 0))
    rep = lambda s: pl.BlockSpec(s, lambda i: (0,) * len(s))
    wmat = rep((EMBED, EMBED))
    wvec = rep((EMBED,))
    return pl.pallas_call(
        _post_body, grid=grid,
        in_specs=[bs1, bsa, bsm, bsm, rep((ANGLE_EMB, EMBED)),
                  wmat, wvec, wmat, wvec, wmat, wvec,
                  wmat, wvec, wmat, wvec,
                  wmat, wvec, wmat, wvec,
                  rep((NUM_RBF, EMBED))],
        out_specs=[bsm, bsm],
        out_shape=[jax.ShapeDtypeStruct((N_EDGES, EMBED), jnp.float32)] * 2,
    )(d3, agg, xji, m, wup,
      rb['W1'], rb['b1'], rb['W2'], rb['b2'], wskip, bskip,
      ra1['W1'], ra1['b1'], ra1['W2'], ra1['b2'],
      ra2['W1'], ra2['b1'], ra2['W2'], ra2['b2'], wnext)


def _outmlp_body(t_ref, wup_ref, w1_ref, b1_ref, w2_ref, b2_ref,
                 w3_ref, b3_ref, wout_ref, o_ref):
    b = pl.program_id(1)

    def dot(a, w):
        return jnp.dot(a, w, preferred_element_type=jnp.float32)

    u = dot(t_ref[0], wup_ref[0])
    u = _act(dot(u, w1_ref[0]) + b1_ref[0])
    u = _act(dot(u, w2_ref[0]) + b2_ref[0])
    u = _act(dot(u, w3_ref[0]) + b3_ref[0])
    contrib = dot(u, wout_ref[0])

    @pl.when(b == 0)
    def _():
        o_ref[...] = contrib

    @pl.when(b > 0)
    def _():
        o_ref[...] += contrib


def _k_outmlp(t_all, wup_s, w1_s, b1_s, w2_s, b2_s, w3_s, b3_s, wout_s):
    nb = N_INTER + 1
    grid = (N_PART // B_T, nb)
    wm = lambda d1, d2: pl.BlockSpec((1, d1, d2), lambda a, b: (b, 0, 0))
    wv = lambda d1: pl.BlockSpec((1, 1, d1), lambda a, b: (b, 0, 0))
    return pl.pallas_call(
        _outmlp_body, grid=grid,
        in_specs=[pl.BlockSpec((1, B_T, EMBED), lambda a, b: (b, a, 0)),
                  wm(EMBED, OUT_EMB), wm(OUT_EMB, OUT_EMB), wv(OUT_EMB),
                  wm(OUT_EMB, OUT_EMB), wv(OUT_EMB),
                  wm(OUT_EMB, OUT_EMB), wv(OUT_EMB), wm(OUT_EMB, EMBED)],
        out_specs=pl.BlockSpec((B_T, EMBED), lambda a, b: (a, 0)),
        out_shape=jax.ShapeDtypeStruct((N_PART, EMBED), jnp.float32),
    )(t_all, wup_s, w1_s, b1_s[:, None, :], w2_s, b2_s[:, None, :],
      w3_s, b3_s[:, None, :], wout_s)


_SC_MESH = None


def _mesh():
    global _SC_MESH
    if _SC_MESH is None:
        _SC_MESH = plsc.VectorSubcoreMesh(core_axis_name="c",
                                          subcore_axis_name="s")
    return _SC_MESH


def _wid():
    return lax.axis_index("s") * NC + lax.axis_index("c")


@functools.lru_cache(maxsize=None)
def _mk_gather(n_chunks, n_rows, width):
    n_out = n_chunks * NW * 128

    @functools.partial(
        pl.kernel, mesh=_mesh(),
        out_type=jax.ShapeDtypeStruct((n_out, width), jnp.float32),
        scratch_types=[pltpu.VMEM((128,), jnp.int32),
                       pltpu.VMEM((128, width), jnp.float32),
                       pltpu.SemaphoreType.DMA],
    )
    def _g(tbl_hbm, idx_hbm, out_hbm, i128, buf, sem):
        w = _wid()

        def body(k, _):
            off = (w * n_chunks + k) * 128
            pltpu.sync_copy(idx_hbm.at[pl.ds(off, 128)], i128)
            pltpu.async_copy(tbl_hbm.at[i128], buf, sem).wait()
            pltpu.sync_copy(buf, out_hbm.at[pl.ds(off, 128)])
            return 0

        lax.fori_loop(0, n_chunks, body, 0)

    return _g


ROWS_TRI = E_SUB + 8
ROWS_AT = T_PER_W + 8


@functools.lru_cache(maxsize=None)
def _mk_tri_scatter():
    @functools.partial(
        pl.kernel, mesh=_mesh(),
        out_type=jax.ShapeDtypeStruct((N_EDGES, ANGLE_EMB), jnp.float32),
        scratch_types=[pltpu.VMEM_SHARED((NS * ROWS_TRI, ANGLE_EMB), jnp.float32),
                       pltpu.VMEM((128, ANGLE_EMB), jnp.float32),
                       pltpu.VMEM((128,), jnp.int32)],
    )
    def _k(t_hbm, li_hbm, zeros_hbm, agg_out, shared, tbuf, li):
        w = _wid()
        slot = lax.axis_index("s")
        for j in range(SUB):
            sid = SUB * w + j
            pltpu.sync_copy(zeros_hbm, shared.at[pl.ds(slot * ROWS_TRI, ROWS_TRI)])

            def body(k, _):
                off = (sid * C_TRI + k) * 128
                pltpu.sync_copy(li_hbm.at[pl.ds(off, 128)], li)
                pltpu.sync_copy(t_hbm.at[pl.ds(off, 128)], tbuf)
                pltpu.sync_copy(tbuf, shared.at[li], add=True)
                return 0

            lax.fori_loop(0, C_TRI, body, 0)
            row0 = w * E_PER_W + j * E_SUB
            pltpu.sync_copy(shared.at[pl.ds(slot * ROWS_TRI, E_SUB)],
                            agg_out.at[pl.ds(row0, E_SUB)])

    return _k


@functools.lru_cache(maxsize=None)
def _mk_atom_scatter():
    @functools.partial(
        pl.kernel, mesh=_mesh(),
        out_type=jax.ShapeDtypeStruct((N_TPAD, EMBED), jnp.float32),
        scratch_types=[pltpu.VMEM_SHARED((NS * ROWS_AT, EMBED), jnp.float32),
                       pltpu.VMEM((128, EMBED), jnp.float32),
                       pltpu.VMEM((128,), jnp.int32),
                       pltpu.VMEM((128,), jnp.int32),
                       pltpu.SemaphoreType.DMA],
    )
    def _k(gm_hbm, pi_hbm, li_hbm, zeros_hbm, t_out,
           shared, buf, pi, li, sem):
        w = _wid()
        slot = lax.axis_index("s")
        pltpu.sync_copy(zeros_hbm, shared.at[pl.ds(slot * ROWS_AT, ROWS_AT)])

        def body(k, _):
            off = (w * C_AT + k) * 128
            pltpu.sync_copy(pi_hbm.at[pl.ds(off, 128)], pi)
            pltpu.async_copy(gm_hbm.at[pi], buf, sem).wait()
            pltpu.sync_copy(li_hbm.at[pl.ds(off, 128)], li)
            pltpu.sync_copy(buf, shared.at[li], add=True)
            return 0

        lax.fori_loop(0, C_AT, body, 0)
        pltpu.sync_copy(shared.at[pl.ds(slot * ROWS_AT, T_PER_W)],
                        t_out.at[pl.ds(w * T_PER_W, T_PER_W)])

    return _k


def kernel(distances, angles, params, species, idx_i, idx_j, angle_mask,
           reduce_to_ji, expand_to_kj):
    i32 = jnp.int32
    f32 = jnp.float32

    perm_a = jnp.argsort(reduce_to_ji).astype(i32)
    r_s = jnp.take(reduce_to_ji, perm_a).astype(i32)
    e_s = jnp.take(expand_to_kj, perm_a).astype(i32)
    sub = r_s // E_SUB
    starts = jnp.searchsorted(sub, jnp.arange(NSUB + 1, dtype=i32)).astype(i32)
    rank = jnp.arange(N_ANGLES, dtype=i32) - jnp.take(starts, sub)
    pos = sub * (C_TRI * 128) + rank
    run = jnp.arange(A2_CAP, dtype=i32) // (C_TRI * 128)
    slot_run = (run // SUB) // NC
    li_a = (slot_run * ROWS_TRI + E_SUB).astype(i32)
    slot_r = (r_s // E_PER_W) // NC
    li_a = li_a.at[pos].set(slot_r * ROWS_TRI + r_s % E_SUB)
    e_pad = jnp.zeros((A2_CAP,), i32).at[pos].set(e_s)
    pa_pad = jnp.zeros((A2_CAP,), i32).at[pos].set(perm_a)

    perm_e = jnp.argsort(idx_i).astype(i32)
    i_s = jnp.take(idx_i, perm_e).astype(i32)
    own = i_s // T_PER_W
    st_e = jnp.searchsorted(own, jnp.arange(33, dtype=i32)).astype(i32)
    rank_e = jnp.arange(N_EDGES, dtype=i32) - jnp.take(st_e, own)
    pos_e = own * (C_AT * 128) + rank_e
    run_e = jnp.arange(E2_CAP, dtype=i32) // (C_AT * 128)
    li_e = ((run_e // NC) * ROWS_AT + T_PER_W).astype(i32)
    li_e = li_e.at[pos_e].set((own // NC) * ROWS_AT + i_s % T_PER_W)
    pe_pad = jnp.zeros((E2_CAP,), i32).at[pos_e].set(perm_e)

    idx_i_p = jnp.concatenate([idx_i.astype(i32),
                               jnp.zeros((EG_CAP - N_EDGES,), i32)])
    idx_j_p = jnp.concatenate([idx_j.astype(i32),
                               jnp.zeros((EG_CAP - N_EDGES,), i32)])

    z_tri = jnp.zeros((ROWS_TRI, ANGLE_EMB), f32)
    z_atom = jnp.zeros((ROWS_AT, EMBED), f32)

    p = params
    embwj = p['emb'] @ p['W_edge'][:TYPE_EMB]
    embwi = p['emb'] @ p['W_edge'][TYPE_EMB:2 * TYPE_EMB]
    wr = p['W_edge'][2 * TYPE_EMB:]
    obs = p['out_blocks']
    wup_s = jnp.stack([ob['W_up'] for ob in obs])
    w1_s = jnp.stack([ob['Ws'][0] for ob in obs])
    w2_s = jnp.stack([ob['Ws'][1] for ob in obs])
    w3_s = jnp.stack([ob['Ws'][2] for ob in obs])
    b1_s = jnp.stack([ob['bs'][0] for ob in obs])
    b2_s = jnp.stack([ob['bs'][1] for ob in obs])
    b3_s = jnp.stack([ob['bs'][2] for ob in obs])
    wout_s = jnp.stack([jnp.pad(ob['W_out'], ((0, 0), (0, EMBED - NUM_TARGETS)))
                        for ob in obs])

    g_e = _mk_gather(EG_CH, N_PART, EMBED)
    g_a = _mk_gather(A2_CH, N_ANGLES, EMBED)
    k_tri = _mk_tri_scatter()
    k_atom = _mk_atom_scatter()

    sp3 = species.astype(i32).reshape(N_PART // B_SP, 1, B_SP)
    cj, ci = _k_ctab(sp3, embwj, embwi)
    cgj = g_e(cj, idx_j_p)[:N_EDGES]
    cgi = g_e(ci, idx_i_p)[:N_EDGES]

    ang3 = angles.reshape(N_ANGLES // B_CA, 1, B_CA)
    mask3 = angle_mask.astype(f32).reshape(N_ANGLES // B_CA, 1, B_CA)
    cbf128 = _k_cbf128(ang3, mask3)
    cbf_g = g_a(cbf128, pa_pad)

    d3 = distances.reshape(N_EDGES // B_E, 1, B_E)
    rad128 = _k_rad128(d3)
    rad_g = g_a(rad128, e_pad)

    m, gm = _k_embed(d3, cgj, cgi, wr, p['b_edge'],
                     p['W_rbf_emb'], p['b_rbf_emb'], obs[0]['W_rbf'])
    t_list = [k_atom(gm, pe_pad, li_e, z_atom)]
    for i in range(N_INTER):
        ib = p['int_blocks'][i]
        wrbf12 = ib['W_rbf1'] @ ib['W_rbf2']
        w48 = jnp.pad(ib['W_sbf1'] @ ib['W_sbf2'], ((0, 6), (0, 0)))
        xji, xdown = _k_pre(d3, m, ib['W_ji'], ib['b_ji'],
                            ib['W_kj'], ib['b_kj'], wrbf12, ib['W_down'])
        x_g = g_a(xdown, e_pad)
        t = _k_t(rad_g, cbf_g, x_g, w48)
        agg = k_tri(t, li_a, z_tri)
        m, gm = _k_post(d3, agg, xji, m, ib['W_up'],
                        ib['res_before'][0], ib['W_skip'], ib['b_skip'],
                        ib['res_after'][0], ib['res_after'][1],
                        obs[i + 1]['W_rbf'])
        t_list.append(k_atom(gm, pe_pad, li_e, z_atom))

    t_all = jnp.stack([t[:N_PART] for t in t_list])
    out_pad = _k_outmlp(t_all, wup_s, w1_s, b1_s, w2_s, b2_s, w3_s, b3_s, wout_s)
    return out_pad[:, :NUM_TARGETS]
